# baseline (speedup 1.0000x reference)
import math

import jax
import jax.numpy as jnp
from jax.experimental import pallas as pl
from jax.experimental.pallas import tpu as pltpu


_LAMDA = 1.5
_ALPHA = 0.7


def _body(x_ref, adj_ref, w0_ref, b0_ref, w1_ref, w2_ref, w3_ref, out_ref,
          adj_s, w0_s, wp_s):
    bf = jnp.bfloat16
    H = w0_ref.shape[1]
    thetas = tuple(min(1.0, math.log(_LAMDA / i + 1.0)) for i in (1, 2, 3))

    @pl.when(pl.program_id(0) == 0)
    def _fold():
        adj_s[...] = ((1.0 - _ALPHA) * adj_ref[...]).astype(bf)
        w0_s[...] = w0_ref[...].astype(bf)
        row = jax.lax.broadcasted_iota(jnp.int32, (H, H), 0)
        col = jax.lax.broadcasted_iota(jnp.int32, (H, H), 1)
        eye = jnp.where(row == col, 1.0, 0.0)
        for i, (th, w_ref) in enumerate(zip(thetas, (w1_ref, w2_ref, w3_ref))):
            wp_s[i] = (th * w_ref[...] + (1.0 - th) * eye).astype(bf)

    N = adj_ref.shape[0]
    nmb = 4
    MB = N // nmb
    w0 = w0_s[...]
    b0v = b0_ref[...]
    hb = [
        jnp.maximum(
            jnp.dot(x_ref[0, mb * MB:(mb + 1) * MB].astype(bf), w0,
                    preferred_element_type=jnp.float32)
            + b0v,
            0.0,
        )
        for mb in range(nmb)
    ]
    ah0b = [_ALPHA * h for h in hb]
    for i in range(3):
        wp = wp_s[i]
        h_bf = jnp.concatenate([h.astype(bf) for h in hb], axis=0)
        for mb in range(nmb):
            sup = (
                jnp.dot(adj_s[mb * MB:(mb + 1) * MB], h_bf,
                        preferred_element_type=jnp.float32)
                + ah0b[mb]
            )
            hb[mb] = jnp.maximum(
                jnp.dot(sup.astype(bf), wp, preferred_element_type=jnp.float32)
                + hb[mb],
                0.0,
            )
    for mb in range(nmb):
        out_ref[0, mb * MB:(mb + 1) * MB] = hb[mb]


def kernel(x, adj, W0, b0, W1, W2, W3):
    B, N, F = x.shape
    H = W0.shape[1]
    b0_2d = b0.reshape(1, H)
    nb = 1
    in_specs = [
            pl.BlockSpec((nb, N, F), lambda b: (b, 0, 0)),
            pl.BlockSpec((N, N), lambda b: (0, 0)),
            pl.BlockSpec((F, H), lambda b: (0, 0)),
            pl.BlockSpec((1, H), lambda b: (0, 0)),
            pl.BlockSpec((H, H), lambda b: (0, 0)),
            pl.BlockSpec((H, H), lambda b: (0, 0)),
            pl.BlockSpec((H, H), lambda b: (0, 0)),
    ]
    return pl.pallas_call(
        _body,
        grid=(B // nb,),
        in_specs=in_specs,
        out_specs=pl.BlockSpec((nb, N, H), lambda b: (b, 0, 0)),
        out_shape=jax.ShapeDtypeStruct((B, N, H), jnp.float32),
        scratch_shapes=[
            pltpu.VMEM((N, N), jnp.bfloat16),
            pltpu.VMEM((F, H), jnp.bfloat16),
            pltpu.VMEM((3, H, H), jnp.bfloat16),
        ],
    )(x, adj, W0, b0_2d, W1, W2, W3)


# speedup vs baseline: 1.1510x; 1.1510x over previous
import math

import jax
import jax.numpy as jnp
from jax.experimental import pallas as pl
from jax.experimental.pallas import tpu as pltpu


_LAMDA = 1.5
_ALPHA = 0.7


def _body(x_ref, adj_ref, w0_ref, b0_ref, w1_ref, w2_ref, w3_ref, out_ref,
          adj_s, w0_s, wp_s, h_s):
    bf = jnp.bfloat16
    H = w0_ref.shape[1]
    thetas = tuple(min(1.0, math.log(_LAMDA / i + 1.0)) for i in (1, 2, 3))

    @pl.when(pl.program_id(0) == 0)
    def _fold():
        adj_s[...] = ((1.0 - _ALPHA) * adj_ref[...]).astype(bf)
        w0_s[...] = w0_ref[...].astype(bf)
        row = jax.lax.broadcasted_iota(jnp.int32, (H, H), 0)
        col = jax.lax.broadcasted_iota(jnp.int32, (H, H), 1)
        eye = jnp.where(row == col, 1.0, 0.0)
        for i, (th, w_ref) in enumerate(zip(thetas, (w1_ref, w2_ref, w3_ref))):
            wp_s[i] = (th * w_ref[...] + (1.0 - th) * eye).astype(bf)

    nb = x_ref.shape[0]
    H = w0_ref.shape[1]
    w0 = w0_s[...]
    b0v = b0_ref[...]
    for b in range(nb):
        h_s[:, b * H:(b + 1) * H] = jnp.maximum(
            jnp.dot(x_ref[b].astype(bf), w0,
                    preferred_element_type=jnp.float32)
            + b0v,
            0.0,
        )
    h_all = h_s[...]
    ah0 = _ALPHA * h_all
    adj = adj_s[...]
    for i in range(3):
        wp = wp_s[i]
        sup = (
            jnp.dot(adj, h_all.astype(bf), preferred_element_type=jnp.float32)
            + ah0
        )
        for b in range(nb):
            sl = slice(b * H, (b + 1) * H)
            h_s[:, sl] = jnp.maximum(
                jnp.dot(sup[:, sl].astype(bf), wp,
                        preferred_element_type=jnp.float32)
                + h_all[:, sl],
                0.0,
            )
        h_all = h_s[...]
    for b in range(nb):
        out_ref[b] = h_all[:, b * H:(b + 1) * H]


def kernel(x, adj, W0, b0, W1, W2, W3):
    B, N, F = x.shape
    H = W0.shape[1]
    b0_2d = b0.reshape(1, H)
    nb = 2
    in_specs = [
            pl.BlockSpec((nb, N, F), lambda b: (b, 0, 0)),
            pl.BlockSpec((N, N), lambda b: (0, 0)),
            pl.BlockSpec((F, H), lambda b: (0, 0)),
            pl.BlockSpec((1, H), lambda b: (0, 0)),
            pl.BlockSpec((H, H), lambda b: (0, 0)),
            pl.BlockSpec((H, H), lambda b: (0, 0)),
            pl.BlockSpec((H, H), lambda b: (0, 0)),
    ]
    return pl.pallas_call(
        _body,
        grid=(B // nb,),
        in_specs=in_specs,
        out_specs=pl.BlockSpec((nb, N, H), lambda b: (b, 0, 0)),
        out_shape=jax.ShapeDtypeStruct((B, N, H), jnp.float32),
        scratch_shapes=[
            pltpu.VMEM((N, N), jnp.bfloat16),
            pltpu.VMEM((F, H), jnp.bfloat16),
            pltpu.VMEM((3, H, H), jnp.bfloat16),
            pltpu.VMEM((N, nb * H), jnp.float32),
        ],
    )(x, adj, W0, b0_2d, W1, W2, W3)


# speedup vs baseline: 1.1734x; 1.0195x over previous
import math

import jax
import jax.numpy as jnp
from jax.experimental import pallas as pl
from jax.experimental.pallas import tpu as pltpu


_LAMDA = 1.5
_ALPHA = 0.7


def _body(x_ref, adj_ref, w0_ref, b0_ref, w1_ref, w2_ref, w3_ref, out_ref,
          adj_s, w0_s, wp_s):
    bf = jnp.bfloat16
    H = w0_ref.shape[1]
    thetas = tuple(min(1.0, math.log(_LAMDA / i + 1.0)) for i in (1, 2, 3))

    @pl.when(pl.program_id(0) == 0)
    def _fold():
        adj_s[...] = ((1.0 - _ALPHA) * adj_ref[...]).astype(bf)
        w0_s[...] = w0_ref[...].astype(bf)
        row = jax.lax.broadcasted_iota(jnp.int32, (H, H), 0)
        col = jax.lax.broadcasted_iota(jnp.int32, (H, H), 1)
        eye = jnp.where(row == col, 1.0, 0.0)
        for i, (th, w_ref) in enumerate(zip(thetas, (w1_ref, w2_ref, w3_ref))):
            wp_s[i] = (th * w_ref[...] + (1.0 - th) * eye).astype(bf)

    h = jnp.maximum(
        jnp.dot(x_ref[0].astype(bf), w0_s[...],
                preferred_element_type=jnp.float32)
        + b0_ref[...],
        0.0,
    )
    ah0 = _ALPHA * h
    adj = adj_s[...]
    for i in range(3):
        sup = jnp.dot(adj, h.astype(bf), preferred_element_type=jnp.float32) + ah0
        h = jnp.maximum(
            jnp.dot(sup.astype(bf), wp_s[i], preferred_element_type=jnp.float32)
            + h,
            0.0,
        )
    out_ref[0] = h


def kernel(x, adj, W0, b0, W1, W2, W3):
    B, N, F = x.shape
    H = W0.shape[1]
    b0_2d = b0.reshape(1, H)
    nb = 1
    in_specs = [
            pl.BlockSpec((nb, N, F), lambda b: (b, 0, 0)),
            pl.BlockSpec((N, N), lambda b: (0, 0)),
            pl.BlockSpec((F, H), lambda b: (0, 0)),
            pl.BlockSpec((1, H), lambda b: (0, 0)),
            pl.BlockSpec((H, H), lambda b: (0, 0)),
            pl.BlockSpec((H, H), lambda b: (0, 0)),
            pl.BlockSpec((H, H), lambda b: (0, 0)),
    ]
    return pl.pallas_call(
        _body,
        grid=(B // nb,),
        in_specs=in_specs,
        out_specs=pl.BlockSpec((nb, N, H), lambda b: (b, 0, 0)),
        out_shape=jax.ShapeDtypeStruct((B, N, H), jnp.float32),
        scratch_shapes=[
            pltpu.VMEM((N, N), jnp.bfloat16),
            pltpu.VMEM((F, H), jnp.bfloat16),
            pltpu.VMEM((3, H, H), jnp.bfloat16),
        ],
    )(x, adj, W0, b0_2d, W1, W2, W3)
